# Initial kernel scaffold; baseline (speedup 1.0000x reference)
import jax
import jax.numpy as jnp
from jax.experimental import pallas as pl


def kernel(feat_table, W1, b1, nodes, neigh_index):
    raise NotImplementedError("write your pallas kernel here")


# speedup vs baseline: 1.2292x; 1.2292x over previous
import functools

import jax
import jax.numpy as jnp
from jax import lax
from jax.experimental import pallas as pl
from jax.experimental.pallas import tpu as pltpu
from jax.experimental.pallas import tpu_sc as plsc

N_NODES = 10000
DEG = 32
D = 128
B = 10000

NC = 2
NS = 16
NW = NC * NS
LANES = 16

BPAD = 10240
RPW = BPAD // NW
RCHUNK = 8
NCHUNKS = RPW // RCHUNK


def _tc_project(feat_table, wx, b1row):
    def body(f_ref, w_ref, b_ref, ps_ref, pn_ref):
        f = f_ref[...]
        w = w_ref[...]
        ps_ref[...] = (
            jnp.dot(f, w[:, :D], preferred_element_type=jnp.float32) + b_ref[...]
        )
        pn_ref[...] = jnp.dot(f, w[:, D:], preferred_element_type=jnp.float32)

    blk = 1000
    return pl.pallas_call(
        body,
        grid=(N_NODES // blk,),
        in_specs=[
            pl.BlockSpec((blk, D), lambda i: (i, 0)),
            pl.BlockSpec((D, 2 * D), lambda i: (0, 0)),
            pl.BlockSpec((1, D), lambda i: (0, 0)),
        ],
        out_specs=[
            pl.BlockSpec((blk, D), lambda i: (i, 0)),
            pl.BlockSpec((blk, D), lambda i: (i, 0)),
        ],
        out_shape=[
            jax.ShapeDtypeStruct((N_NODES, D), jnp.float32),
            jax.ShapeDtypeStruct((N_NODES, D), jnp.float32),
        ],
    )(feat_table, wx, b1row)


def _make_sc_gather_sum():
    mesh = plsc.VectorSubcoreMesh(core_axis_name="c", subcore_axis_name="s")

    @functools.partial(
        pl.kernel,
        mesh=mesh,
        out_type=jax.ShapeDtypeStruct((BPAD, D), jnp.float32),
        scratch_types=[
            pltpu.VMEM((RCHUNK,), jnp.int32),
            pltpu.VMEM((RCHUNK * DEG,), jnp.int32),
            pltpu.VMEM((RCHUNK, D), jnp.float32),
            pltpu.VMEM((RCHUNK * DEG, D), jnp.float32),
            pltpu.VMEM((RCHUNK, D), jnp.float32),
            pltpu.SemaphoreType.DMA,
            pltpu.SemaphoreType.DMA,
        ],
    )
    def sc_kernel(ps_hbm, pn_hbm, nodes_hbm, neigh_hbm, out_hbm,
                  idxs_v, idxn_v, rows_s, rows_n, out_v, sem_s, sem_n):
        wid = lax.axis_index("s") * NC + lax.axis_index("c")
        base = wid * RPW

        def chunk(k, carry):
            b0 = base + k * RCHUNK
            pltpu.sync_copy(nodes_hbm.at[pl.ds(b0, RCHUNK)], idxs_v)
            pltpu.sync_copy(neigh_hbm.at[pl.ds(b0 * DEG, RCHUNK * DEG)], idxn_v)
            cs = pltpu.async_copy(ps_hbm.at[idxs_v], rows_s, sem_s)
            cn0 = pltpu.async_copy(
                pn_hbm.at[idxn_v.at[pl.ds(0, 128)]],
                rows_n.at[pl.ds(0, 128)], sem_n)
            cn1 = pltpu.async_copy(
                pn_hbm.at[idxn_v.at[pl.ds(128, 128)]],
                rows_n.at[pl.ds(128, 128)], sem_n)
            cs.wait()
            cn0.wait()
            cn1.wait()

            def row(r, carry2):
                for c in range(D // LANES):
                    sl = pl.ds(c * LANES, LANES)
                    a0 = rows_s[r, sl] + rows_n[r * DEG + 0, sl]
                    a1 = rows_n[r * DEG + 1, sl]
                    a2 = rows_n[r * DEG + 2, sl]
                    a3 = rows_n[r * DEG + 3, sl]
                    for j in range(4, DEG, 4):
                        a0 = a0 + rows_n[r * DEG + j, sl]
                        a1 = a1 + rows_n[r * DEG + j + 1, sl]
                        a2 = a2 + rows_n[r * DEG + j + 2, sl]
                        a3 = a3 + rows_n[r * DEG + j + 3, sl]
                    acc = (a0 + a1) + (a2 + a3)
                    out_v[r, sl] = jnp.maximum(acc, 0.0)
                return carry2

            lax.fori_loop(0, RCHUNK, row, 0)
            pltpu.sync_copy(out_v, out_hbm.at[pl.ds(b0, RCHUNK)])
            return carry

        lax.fori_loop(0, NCHUNKS, chunk, 0)

    return sc_kernel


def kernel(feat_table, W1, b1, nodes, neigh_index):
    wt = W1.T.astype(jnp.float32)
    wx = jnp.concatenate([wt[:D, :], wt[D:, :] * (1.0 / DEG)], axis=1)
    b1row = b1.astype(jnp.float32).reshape(1, D)

    p_self, p_neigh = _tc_project(feat_table.astype(jnp.float32), wx, b1row)

    nodes_p = jnp.concatenate(
        [nodes.astype(jnp.int32), jnp.zeros((BPAD - B,), jnp.int32)])
    neigh_p = jnp.concatenate(
        [neigh_index.astype(jnp.int32).reshape(-1),
         jnp.zeros(((BPAD - B) * DEG,), jnp.int32)])

    out = _make_sc_gather_sum()(p_self, p_neigh, nodes_p, neigh_p)
    return out[:B]


# speedup vs baseline: 1.3901x; 1.1309x over previous
import functools

import jax
import jax.numpy as jnp
from jax import lax
from jax.experimental import pallas as pl
from jax.experimental.pallas import tpu as pltpu
from jax.experimental.pallas import tpu_sc as plsc

N_NODES = 10000
DEG = 32
D = 128
B = 10000

NC = 2
NS = 16
NW = NC * NS
LANES = 16

BPAD = 10240
RPW = BPAD // NW
RCHUNK = 8
NCHUNKS = RPW // RCHUNK
NGI = RCHUNK * DEG // 128


def _tc_project(feat_table, wx, b1row):
    def body(f_ref, w_ref, b_ref, ps_ref, pn_ref):
        f = f_ref[...]
        w = w_ref[...]
        ps_ref[...] = (
            jnp.dot(f, w[:, :D], preferred_element_type=jnp.float32) + b_ref[...]
        )
        pn_ref[...] = jnp.dot(f, w[:, D:], preferred_element_type=jnp.float32)

    blk = 1000
    return pl.pallas_call(
        body,
        grid=(N_NODES // blk,),
        in_specs=[
            pl.BlockSpec((blk, D), lambda i: (i, 0)),
            pl.BlockSpec((D, 2 * D), lambda i: (0, 0)),
            pl.BlockSpec((1, D), lambda i: (0, 0)),
        ],
        out_specs=[
            pl.BlockSpec((blk, D), lambda i: (i, 0)),
            pl.BlockSpec((blk, D), lambda i: (i, 0)),
        ],
        out_shape=[
            jax.ShapeDtypeStruct((N_NODES, D), jnp.float32),
            jax.ShapeDtypeStruct((N_NODES, D), jnp.float32),
        ],
    )(feat_table, wx, b1row)


def _make_sc_gather_sum():
    mesh = plsc.VectorSubcoreMesh(core_axis_name="c", subcore_axis_name="s")

    @functools.partial(
        pl.kernel,
        mesh=mesh,
        out_type=jax.ShapeDtypeStruct((BPAD, D), jnp.float32),
        scratch_types=[
            pltpu.VMEM((RPW,), jnp.int32),
            pltpu.VMEM((RPW * DEG,), jnp.int32),
            pltpu.VMEM((RCHUNK, D), jnp.float32),
            pltpu.VMEM((RCHUNK, D), jnp.float32),
            pltpu.VMEM((RCHUNK * DEG, D), jnp.float32),
            pltpu.VMEM((RCHUNK * DEG, D), jnp.float32),
            pltpu.VMEM((RCHUNK, D), jnp.float32),
            pltpu.VMEM((RCHUNK, D), jnp.float32),
            pltpu.SemaphoreType.DMA,
            pltpu.SemaphoreType.DMA,
            pltpu.SemaphoreType.DMA,
            pltpu.SemaphoreType.DMA,
        ],
    )
    def sc_kernel(ps_hbm, pn_hbm, nodes_hbm, neigh_hbm, out_hbm,
                  idxs_all, idxn_all, rs0, rs1, rn0, rn1, ov0, ov1,
                  gsem0, gsem1, osem0, osem1):
        wid = lax.axis_index("s") * NC + lax.axis_index("c")
        base = wid * RPW

        rows_s = (rs0, rs1)
        rows_n = (rn0, rn1)
        out_v = (ov0, ov1)
        gsem = (gsem0, gsem1)
        osem = (osem0, osem1)

        pltpu.sync_copy(nodes_hbm.at[pl.ds(base, RPW)], idxs_all)
        pltpu.sync_copy(neigh_hbm.at[pl.ds(base * DEG, RPW * DEG)], idxn_all)

        def issue(c, slot):
            pltpu.async_copy(
                ps_hbm.at[idxs_all.at[pl.ds(c * RCHUNK, RCHUNK)]],
                rows_s[slot], gsem[slot])
            for j in range(NGI):
                pltpu.async_copy(
                    pn_hbm.at[idxn_all.at[pl.ds(c * (RCHUNK * DEG) + j * 128, 128)]],
                    rows_n[slot].at[pl.ds(j * 128, 128)], gsem[slot])

        def wait_gathers(slot):
            pltpu.make_async_copy(
                ps_hbm.at[pl.ds(0, RCHUNK)], rows_s[slot], gsem[slot]).wait()
            for j in range(NGI):
                pltpu.make_async_copy(
                    pn_hbm.at[pl.ds(0, 128)],
                    rows_n[slot].at[pl.ds(j * 128, 128)], gsem[slot]).wait()

        def compute(slot):
            rs = rows_s[slot]
            rn = rows_n[slot]
            ov = out_v[slot]

            def row(r, carry2):
                for c in range(D // LANES):
                    sl = pl.ds(c * LANES, LANES)
                    a0 = rs[r, sl] + rn[r * DEG + 0, sl]
                    a1 = rn[r * DEG + 1, sl]
                    a2 = rn[r * DEG + 2, sl]
                    a3 = rn[r * DEG + 3, sl]
                    for j in range(4, DEG, 4):
                        a0 = a0 + rn[r * DEG + j, sl]
                        a1 = a1 + rn[r * DEG + j + 1, sl]
                        a2 = a2 + rn[r * DEG + j + 2, sl]
                        a3 = a3 + rn[r * DEG + j + 3, sl]
                    acc = (a0 + a1) + (a2 + a3)
                    ov[r, sl] = jnp.maximum(acc, 0.0)
                return carry2

            lax.fori_loop(0, RCHUNK, row, 0)

        def step(c, i, slot):
            wait_gathers(slot)

            @pl.when(c + 1 < NCHUNKS)
            def _():
                issue(c + 1, 1 - slot)

            @pl.when(i > 0)
            def _():
                pltpu.make_async_copy(
                    out_v[slot], out_hbm.at[pl.ds(0, RCHUNK)], osem[slot]).wait()

            compute(slot)
            pltpu.async_copy(
                out_v[slot], out_hbm.at[pl.ds(base + c * RCHUNK, RCHUNK)],
                osem[slot])

        issue(0, 0)

        def pair(i, carry):
            step(2 * i, i, 0)
            step(2 * i + 1, i, 1)
            return carry

        lax.fori_loop(0, NCHUNKS // 2, pair, 0)

        pltpu.make_async_copy(
            out_v[0], out_hbm.at[pl.ds(0, RCHUNK)], osem[0]).wait()
        pltpu.make_async_copy(
            out_v[1], out_hbm.at[pl.ds(0, RCHUNK)], osem[1]).wait()

    return sc_kernel


def kernel(feat_table, W1, b1, nodes, neigh_index):
    wt = W1.T.astype(jnp.float32)
    wx = jnp.concatenate([wt[:D, :], wt[D:, :] * (1.0 / DEG)], axis=1)
    b1row = b1.astype(jnp.float32).reshape(1, D)

    p_self, p_neigh = _tc_project(feat_table.astype(jnp.float32), wx, b1row)

    nodes_p = jnp.concatenate(
        [nodes.astype(jnp.int32), jnp.zeros((BPAD - B,), jnp.int32)])
    neigh_p = jnp.concatenate(
        [neigh_index.astype(jnp.int32).reshape(-1),
         jnp.zeros(((BPAD - B) * DEG,), jnp.int32)])

    out = _make_sc_gather_sum()(p_self, p_neigh, nodes_p, neigh_p)
    return out[:B]


# speedup vs baseline: 5.4852x; 3.9460x over previous
import functools

import jax
import jax.numpy as jnp
from jax import lax
from jax.experimental import pallas as pl
from jax.experimental.pallas import tpu as pltpu
from jax.experimental.pallas import tpu_sc as plsc

N_NODES = 10000
DEG = 32
D = 128
B = 10000

NC = 2
NS = 16
NW = NC * NS
LANES = 16

NPAD = 10112
SROWS = NPAD // NS
BPAD = 10240
RPW = BPAD // NW
RCHUNK = 4
NCHUNKS = RPW // RCHUNK
NPAIRS = NCHUNKS // 2


def _tc_project(feat_table, wx, b1row):
    def body(f_ref, w_ref, b_ref, ps_ref, pn_ref):
        f = f_ref[...]
        w = w_ref[...]
        ps_ref[...] = (
            jnp.dot(f, w[:, :D], preferred_element_type=jnp.float32) + b_ref[...]
        )
        pn_ref[...] = jnp.dot(f, w[:, D:], preferred_element_type=jnp.float32)

    blk = 1264
    return pl.pallas_call(
        body,
        grid=(NPAD // blk,),
        in_specs=[
            pl.BlockSpec((blk, D), lambda i: (i, 0)),
            pl.BlockSpec((D, 2 * D), lambda i: (0, 0)),
            pl.BlockSpec((1, D), lambda i: (0, 0)),
        ],
        out_specs=[
            pl.BlockSpec((blk, D), lambda i: (i, 0)),
            pl.BlockSpec((blk, D), lambda i: (i, 0)),
        ],
        out_shape=[
            jax.ShapeDtypeStruct((NPAD, D), jnp.float32),
            jax.ShapeDtypeStruct((NPAD, D), jnp.float32),
        ],
    )(feat_table, wx, b1row)


def _make_sc_gather_sum():
    mesh = plsc.VectorSubcoreMesh(core_axis_name="c", subcore_axis_name="s")

    @functools.partial(
        pl.kernel,
        mesh=mesh,
        out_type=jax.ShapeDtypeStruct((BPAD, D), jnp.float32),
        scratch_types=[
            pltpu.VMEM_SHARED((NPAD, D), jnp.float32),
            pltpu.VMEM((RPW,), jnp.int32),
            pltpu.VMEM((RPW * DEG,), jnp.int32),
            pltpu.VMEM((2 * RCHUNK, D), jnp.float32),
            pltpu.VMEM((2 * RCHUNK, D), jnp.float32),
            pltpu.VMEM((RCHUNK * DEG, D), jnp.float32),
            pltpu.VMEM((RCHUNK * DEG, D), jnp.float32),
            pltpu.VMEM((RCHUNK, D), jnp.float32),
            pltpu.VMEM((RCHUNK, D), jnp.float32),
            pltpu.SemaphoreType.DMA,
            pltpu.SemaphoreType.DMA,
            pltpu.SemaphoreType.DMA,
            pltpu.SemaphoreType.DMA,
            pltpu.SemaphoreType.DMA,
        ],
    )
    def sc_kernel(ps_hbm, pn_hbm, nodes_hbm, neigh_hbm, out_hbm,
                  shared_tbl, idxs_all, idxn_all, rs0, rs1, rn0, rn1, ov0, ov1,
                  ssem, nsem0, nsem1, osem0, osem1):
        cid = lax.axis_index("c")
        sid = lax.axis_index("s")
        wid = sid * NC + cid
        base = wid * RPW

        rows_s = (rs0, rs1)
        rows_n = (rn0, rn1)
        out_v = (ov0, ov1)
        nsem = (nsem0, nsem1)
        osem = (osem0, osem1)

        pltpu.sync_copy(pn_hbm.at[pl.ds(sid * SROWS, SROWS)],
                        shared_tbl.at[pl.ds(sid * SROWS, SROWS)])
        pltpu.sync_copy(nodes_hbm.at[pl.ds(base, RPW)], idxs_all)
        pltpu.sync_copy(neigh_hbm.at[pl.ds(base * DEG, RPW * DEG)], idxn_all)
        plsc.subcore_barrier()

        def issue_self(p, pslot):
            pltpu.async_copy(
                ps_hbm.at[idxs_all.at[pl.ds(p * 2 * RCHUNK, 2 * RCHUNK)]],
                rows_s[pslot], ssem)

        def wait_self(pslot):
            pltpu.make_async_copy(
                ps_hbm.at[pl.ds(0, 2 * RCHUNK)], rows_s[pslot], ssem).wait()

        def issue_neigh(c, slot):
            pltpu.async_copy(
                shared_tbl.at[idxn_all.at[pl.ds(c * (RCHUNK * DEG), RCHUNK * DEG)]],
                rows_n[slot], nsem[slot])

        def wait_neigh(slot):
            pltpu.make_async_copy(
                pn_hbm.at[pl.ds(0, RCHUNK * DEG)], rows_n[slot],
                nsem[slot]).wait()

        def compute_chunk(nslot, pslot, srow0):
            rn = rows_n[nslot]
            rs = rows_s[pslot]
            ov = out_v[nslot]

            def row(r, carry2):
                for c in range(D // LANES):
                    sl = pl.ds(c * LANES, LANES)
                    a0 = rs[srow0 + r, sl] + rn[r * DEG + 0, sl]
                    a1 = rn[r * DEG + 1, sl]
                    a2 = rn[r * DEG + 2, sl]
                    a3 = rn[r * DEG + 3, sl]
                    for j in range(4, DEG, 4):
                        a0 = a0 + rn[r * DEG + j, sl]
                        a1 = a1 + rn[r * DEG + j + 1, sl]
                        a2 = a2 + rn[r * DEG + j + 2, sl]
                        a3 = a3 + rn[r * DEG + j + 3, sl]
                    acc = (a0 + a1) + (a2 + a3)
                    ov[r, sl] = jnp.maximum(acc, 0.0)
                return carry2

            lax.fori_loop(0, RCHUNK, row, 0)

        def step(c, i, nslot, pslot, srow0, issue_self_next):
            wait_neigh(nslot)

            @pl.when(c + 1 < NCHUNKS)
            def _():
                issue_neigh(c + 1, 1 - nslot)

            if issue_self_next:
                @pl.when(i + 1 < NPAIRS)
                def _():
                    issue_self(i + 1, 1 - pslot)

            @pl.when(i > 0)
            def _():
                pltpu.make_async_copy(
                    out_v[nslot], out_hbm.at[pl.ds(0, RCHUNK)],
                    osem[nslot]).wait()

            compute_chunk(nslot, pslot, srow0)
            pltpu.async_copy(
                out_v[nslot], out_hbm.at[pl.ds(base + c * RCHUNK, RCHUNK)],
                osem[nslot])

        issue_self(0, 0)
        issue_neigh(0, 0)

        def pair2(i2, carry):
            p0 = 2 * i2
            p1 = 2 * i2 + 1
            wait_self(0)
            step(2 * p0, p0, 0, 0, 0, issue_self_next=False)
            step(2 * p0 + 1, p0, 1, 0, RCHUNK, issue_self_next=True)
            wait_self(1)
            step(2 * p1, p1, 0, 1, 0, issue_self_next=False)
            step(2 * p1 + 1, p1, 1, 1, RCHUNK, issue_self_next=True)
            return carry

        lax.fori_loop(0, NPAIRS // 2, pair2, 0)

        pltpu.make_async_copy(
            out_v[0], out_hbm.at[pl.ds(0, RCHUNK)], osem[0]).wait()
        pltpu.make_async_copy(
            out_v[1], out_hbm.at[pl.ds(0, RCHUNK)], osem[1]).wait()

    return sc_kernel


def kernel(feat_table, W1, b1, nodes, neigh_index):
    wt = W1.T.astype(jnp.float32)
    wx = jnp.concatenate([wt[:D, :], wt[D:, :] * (1.0 / DEG)], axis=1)
    b1row = b1.astype(jnp.float32).reshape(1, D)

    feat_p = jnp.concatenate(
        [feat_table.astype(jnp.float32),
         jnp.zeros((NPAD - N_NODES, D), jnp.float32)])
    p_self, p_neigh = _tc_project(feat_p, wx, b1row)

    nodes_p = jnp.concatenate(
        [nodes.astype(jnp.int32), jnp.zeros((BPAD - B,), jnp.int32)])
    neigh_p = jnp.concatenate(
        [neigh_index.astype(jnp.int32).reshape(-1),
         jnp.zeros(((BPAD - B) * DEG,), jnp.int32)])

    out = _make_sc_gather_sum()(p_self, p_neigh, nodes_p, neigh_p)
    return out[:B]


# speedup vs baseline: 5.9649x; 1.0874x over previous
import functools

import jax
import jax.numpy as jnp
from jax import lax
from jax.experimental import pallas as pl
from jax.experimental.pallas import tpu as pltpu
from jax.experimental.pallas import tpu_sc as plsc

N_NODES = 10000
DEG = 32
D = 128
B = 10000

NC = 2
NS = 16
NW = NC * NS
LANES = 16

NPAD = 10112
SROWS = NPAD // NS
RPW = 320
RCHUNK = 4
NCHUNKS = RPW // RCHUNK
NPAIRS = NCHUNKS // 2


def _tc_project(feat_table, wx, b1row):
    def body(f_ref, w_ref, b_ref, ps_ref, pn_ref):
        f = f_ref[...]
        w = w_ref[...]
        ps_ref[...] = (
            jnp.dot(f, w[:, :D], preferred_element_type=jnp.float32) + b_ref[...]
        )
        pn_ref[...] = jnp.dot(f, w[:, D:], preferred_element_type=jnp.float32)

    blk = 1264
    return pl.pallas_call(
        body,
        grid=(NPAD // blk,),
        in_specs=[
            pl.BlockSpec((blk, D), lambda i: (i, 0)),
            pl.BlockSpec((D, 2 * D), lambda i: (0, 0)),
            pl.BlockSpec((1, D), lambda i: (0, 0)),
        ],
        out_specs=[
            pl.BlockSpec((blk, D), lambda i: (i, 0)),
            pl.BlockSpec((blk, D), lambda i: (i, 0)),
        ],
        out_shape=[
            jax.ShapeDtypeStruct((NPAD, D), jnp.float32),
            jax.ShapeDtypeStruct((NPAD, D), jnp.float32),
        ],
    )(feat_table, wx, b1row)


def _make_sc_gather_sum():
    mesh = plsc.VectorSubcoreMesh(core_axis_name="c", subcore_axis_name="s")

    @functools.partial(
        pl.kernel,
        mesh=mesh,
        out_type=jax.ShapeDtypeStruct((B, D), jnp.float32),
        scratch_types=[
            pltpu.VMEM_SHARED((NPAD, D), jnp.float32),
            pltpu.VMEM((RPW,), jnp.int32),
            pltpu.VMEM((RPW * DEG,), jnp.int32),
            pltpu.VMEM((2 * RCHUNK, D), jnp.float32),
            pltpu.VMEM((2 * RCHUNK, D), jnp.float32),
            pltpu.VMEM((RCHUNK * DEG, D), jnp.float32),
            pltpu.VMEM((RCHUNK * DEG, D), jnp.float32),
            pltpu.VMEM((RCHUNK, D), jnp.float32),
            pltpu.VMEM((RCHUNK, D), jnp.float32),
            pltpu.SemaphoreType.DMA,
            pltpu.SemaphoreType.DMA,
            pltpu.SemaphoreType.DMA,
            pltpu.SemaphoreType.DMA,
            pltpu.SemaphoreType.DMA,
        ],
    )
    def sc_kernel(ps_hbm, pn_hbm, nodes_hbm, neigh_hbm, out_hbm,
                  shared_tbl, idxs_all, idxn_all, rs0, rs1, rn0, rn1, ov0, ov1,
                  ssem, nsem0, nsem1, osem0, osem1):
        cid = lax.axis_index("c")
        sid = lax.axis_index("s")
        wid = sid * NC + cid
        base = lax.min(wid * RPW, B - RPW)

        rows_s = (rs0, rs1)
        rows_n = (rn0, rn1)
        out_v = (ov0, ov1)
        nsem = (nsem0, nsem1)
        osem = (osem0, osem1)

        pltpu.sync_copy(pn_hbm.at[pl.ds(sid * SROWS, SROWS)],
                        shared_tbl.at[pl.ds(sid * SROWS, SROWS)])
        pltpu.sync_copy(nodes_hbm.at[pl.ds(base, RPW)], idxs_all)
        pltpu.sync_copy(neigh_hbm.at[pl.ds(base * DEG, RPW * DEG)], idxn_all)
        plsc.subcore_barrier()

        def issue_self(p, pslot):
            pltpu.async_copy(
                ps_hbm.at[idxs_all.at[pl.ds(p * 2 * RCHUNK, 2 * RCHUNK)]],
                rows_s[pslot], ssem)

        def wait_self(pslot):
            pltpu.make_async_copy(
                ps_hbm.at[pl.ds(0, 2 * RCHUNK)], rows_s[pslot], ssem).wait()

        def issue_neigh(c, slot):
            pltpu.async_copy(
                shared_tbl.at[idxn_all.at[pl.ds(c * (RCHUNK * DEG), RCHUNK * DEG)]],
                rows_n[slot], nsem[slot])

        def wait_neigh(slot):
            pltpu.make_async_copy(
                pn_hbm.at[pl.ds(0, RCHUNK * DEG)], rows_n[slot],
                nsem[slot]).wait()

        def compute_chunk(nslot, pslot, srow0):
            rn = rows_n[nslot]
            rs = rows_s[pslot]
            ov = out_v[nslot]

            def row(r, carry2):
                for c in range(D // LANES):
                    sl = pl.ds(c * LANES, LANES)
                    a0 = rs[srow0 + r, sl] + rn[r * DEG + 0, sl]
                    a1 = rn[r * DEG + 1, sl]
                    a2 = rn[r * DEG + 2, sl]
                    a3 = rn[r * DEG + 3, sl]
                    for j in range(4, DEG, 4):
                        a0 = a0 + rn[r * DEG + j, sl]
                        a1 = a1 + rn[r * DEG + j + 1, sl]
                        a2 = a2 + rn[r * DEG + j + 2, sl]
                        a3 = a3 + rn[r * DEG + j + 3, sl]
                    acc = (a0 + a1) + (a2 + a3)
                    ov[r, sl] = jnp.maximum(acc, 0.0)
                return carry2

            lax.fori_loop(0, RCHUNK, row, 0)

        def step(c, i, nslot, pslot, srow0, issue_self_next):
            wait_neigh(nslot)

            @pl.when(c + 1 < NCHUNKS)
            def _():
                issue_neigh(c + 1, 1 - nslot)

            if issue_self_next:
                @pl.when(i + 1 < NPAIRS)
                def _():
                    issue_self(i + 1, 1 - pslot)

            @pl.when(i > 0)
            def _():
                pltpu.make_async_copy(
                    out_v[nslot], out_hbm.at[pl.ds(0, RCHUNK)],
                    osem[nslot]).wait()

            compute_chunk(nslot, pslot, srow0)
            pltpu.async_copy(
                out_v[nslot], out_hbm.at[pl.ds(base + c * RCHUNK, RCHUNK)],
                osem[nslot])

        issue_self(0, 0)
        issue_neigh(0, 0)

        def pair2(i2, carry):
            p0 = 2 * i2
            p1 = 2 * i2 + 1
            wait_self(0)
            step(2 * p0, p0, 0, 0, 0, issue_self_next=False)
            step(2 * p0 + 1, p0, 1, 0, RCHUNK, issue_self_next=True)
            wait_self(1)
            step(2 * p1, p1, 0, 1, 0, issue_self_next=False)
            step(2 * p1 + 1, p1, 1, 1, RCHUNK, issue_self_next=True)
            return carry

        lax.fori_loop(0, NPAIRS // 2, pair2, 0)

        pltpu.make_async_copy(
            out_v[0], out_hbm.at[pl.ds(0, RCHUNK)], osem[0]).wait()
        pltpu.make_async_copy(
            out_v[1], out_hbm.at[pl.ds(0, RCHUNK)], osem[1]).wait()

    return sc_kernel


def kernel(feat_table, W1, b1, nodes, neigh_index):
    wt = W1.T.astype(jnp.float32)
    wx = jnp.concatenate([wt[:D, :], wt[D:, :] * (1.0 / DEG)], axis=1)
    b1row = b1.astype(jnp.float32).reshape(1, D)

    p_self, p_neigh = _tc_project(feat_table.astype(jnp.float32), wx, b1row)

    nodes_i = nodes.astype(jnp.int32)
    neigh_i = neigh_index.astype(jnp.int32).reshape(-1)
    return _make_sc_gather_sum()(p_self, p_neigh, nodes_i, neigh_i)


# speedup vs baseline: 6.0365x; 1.0120x over previous
import functools

import jax
import jax.numpy as jnp
from jax import lax
from jax.experimental import pallas as pl
from jax.experimental.pallas import tpu as pltpu
from jax.experimental.pallas import tpu_sc as plsc

N_NODES = 10000
DEG = 32
D = 128
B = 10000

NC = 2
NS = 16
NW = NC * NS
LANES = 16

NPAD = 10112
SROWS = NPAD // NS
RPW = 320
RCHUNK = 4
NCHUNKS = RPW // RCHUNK
NPAIRS = NCHUNKS // 2


def _tc_project(feat_table, wx, b1row):
    def body(f_ref, w_ref, b_ref, ps_ref, pn_ref):
        f = f_ref[...]
        w = w_ref[...]
        ps_ref[...] = (
            jnp.dot(f, w[:, :D], preferred_element_type=jnp.float32) + b_ref[...]
        )
        pn_ref[...] = jnp.dot(f, w[:, D:], preferred_element_type=jnp.float32)

    blk = 1264
    return pl.pallas_call(
        body,
        grid=(NPAD // blk,),
        in_specs=[
            pl.BlockSpec((blk, D), lambda i: (i, 0)),
            pl.BlockSpec((D, 2 * D), lambda i: (0, 0)),
            pl.BlockSpec((1, D), lambda i: (0, 0)),
        ],
        out_specs=[
            pl.BlockSpec((blk, D), lambda i: (i, 0)),
            pl.BlockSpec((blk, D), lambda i: (i, 0)),
        ],
        out_shape=[
            jax.ShapeDtypeStruct((NPAD, D), jnp.float32),
            jax.ShapeDtypeStruct((NPAD, D), jnp.float32),
        ],
    )(feat_table, wx, b1row)


def _make_sc_gather_sum():
    mesh = plsc.VectorSubcoreMesh(core_axis_name="c", subcore_axis_name="s")

    @functools.partial(
        pl.kernel,
        mesh=mesh,
        out_type=jax.ShapeDtypeStruct((B, D), jnp.float32),
        scratch_types=[
            pltpu.VMEM_SHARED((NPAD, D), jnp.float32),
            pltpu.VMEM((RPW,), jnp.int32),
            pltpu.VMEM((RPW * DEG,), jnp.int32),
            pltpu.VMEM((2 * RCHUNK, D), jnp.float32),
            pltpu.VMEM((2 * RCHUNK, D), jnp.float32),
            pltpu.VMEM((RCHUNK * DEG, D), jnp.float32),
            pltpu.VMEM((RCHUNK * DEG, D), jnp.float32),
            pltpu.VMEM((RCHUNK, D), jnp.float32),
            pltpu.VMEM((RCHUNK, D), jnp.float32),
            pltpu.SemaphoreType.DMA,
            pltpu.SemaphoreType.DMA,
            pltpu.SemaphoreType.DMA,
            pltpu.SemaphoreType.DMA,
            pltpu.SemaphoreType.DMA,
        ],
    )
    def sc_kernel(ps_hbm, pn_hbm, nodes_hbm, neigh_hbm, out_hbm,
                  shared_tbl, idxs_all, idxn_all, rs0, rs1, rn0, rn1, ov0, ov1,
                  ssem, nsem0, nsem1, osem0, osem1):
        cid = lax.axis_index("c")
        sid = lax.axis_index("s")
        wid = sid * NC + cid
        base = lax.min(wid * RPW, B - RPW)

        rows_s = (rs0, rs1)
        rows_n = (rn0, rn1)
        out_v = (ov0, ov1)
        nsem = (nsem0, nsem1)
        osem = (osem0, osem1)

        st0 = pltpu.async_copy(pn_hbm.at[pl.ds(sid * SROWS, SROWS)],
                               shared_tbl.at[pl.ds(sid * SROWS, SROWS)], ssem)
        st1 = pltpu.async_copy(nodes_hbm.at[pl.ds(base, RPW)], idxs_all, ssem)
        st2 = pltpu.async_copy(neigh_hbm.at[pl.ds(base * DEG, RPW * DEG)],
                               idxn_all, ssem)
        st0.wait()
        st1.wait()
        st2.wait()
        plsc.subcore_barrier()

        def issue_self(p, pslot):
            pltpu.async_copy(
                ps_hbm.at[idxs_all.at[pl.ds(p * 2 * RCHUNK, 2 * RCHUNK)]],
                rows_s[pslot], ssem)

        def wait_self(pslot):
            pltpu.make_async_copy(
                ps_hbm.at[pl.ds(0, 2 * RCHUNK)], rows_s[pslot], ssem).wait()

        def issue_neigh(c, slot):
            pltpu.async_copy(
                shared_tbl.at[idxn_all.at[pl.ds(c * (RCHUNK * DEG), RCHUNK * DEG)]],
                rows_n[slot], nsem[slot])

        def wait_neigh(slot):
            pltpu.make_async_copy(
                pn_hbm.at[pl.ds(0, RCHUNK * DEG)], rows_n[slot],
                nsem[slot]).wait()

        def compute_chunk(nslot, pslot, srow0):
            rn = rows_n[nslot]
            rs = rows_s[pslot]
            ov = out_v[nslot]

            def row(r, carry2):
                for c in range(D // LANES):
                    sl = pl.ds(c * LANES, LANES)
                    a0 = rs[srow0 + r, sl] + rn[r * DEG + 0, sl]
                    a1 = rn[r * DEG + 1, sl]
                    a2 = rn[r * DEG + 2, sl]
                    a3 = rn[r * DEG + 3, sl]
                    for j in range(4, DEG, 4):
                        a0 = a0 + rn[r * DEG + j, sl]
                        a1 = a1 + rn[r * DEG + j + 1, sl]
                        a2 = a2 + rn[r * DEG + j + 2, sl]
                        a3 = a3 + rn[r * DEG + j + 3, sl]
                    acc = (a0 + a1) + (a2 + a3)
                    ov[r, sl] = jnp.maximum(acc, 0.0)
                return carry2

            lax.fori_loop(0, RCHUNK, row, 0)

        def step(c, i, nslot, pslot, srow0, issue_self_next):
            wait_neigh(nslot)

            @pl.when(c + 1 < NCHUNKS)
            def _():
                issue_neigh(c + 1, 1 - nslot)

            if issue_self_next:
                @pl.when(i + 1 < NPAIRS)
                def _():
                    issue_self(i + 1, 1 - pslot)

            @pl.when(i > 0)
            def _():
                pltpu.make_async_copy(
                    out_v[nslot], out_hbm.at[pl.ds(0, RCHUNK)],
                    osem[nslot]).wait()

            compute_chunk(nslot, pslot, srow0)
            pltpu.async_copy(
                out_v[nslot], out_hbm.at[pl.ds(base + c * RCHUNK, RCHUNK)],
                osem[nslot])

        issue_self(0, 0)
        issue_neigh(0, 0)

        def pair2(i2, carry):
            p0 = 2 * i2
            p1 = 2 * i2 + 1
            wait_self(0)
            step(2 * p0, p0, 0, 0, 0, issue_self_next=False)
            step(2 * p0 + 1, p0, 1, 0, RCHUNK, issue_self_next=True)
            wait_self(1)
            step(2 * p1, p1, 0, 1, 0, issue_self_next=False)
            step(2 * p1 + 1, p1, 1, 1, RCHUNK, issue_self_next=True)
            return carry

        lax.fori_loop(0, NPAIRS // 2, pair2, 0)

        pltpu.make_async_copy(
            out_v[0], out_hbm.at[pl.ds(0, RCHUNK)], osem[0]).wait()
        pltpu.make_async_copy(
            out_v[1], out_hbm.at[pl.ds(0, RCHUNK)], osem[1]).wait()

    return sc_kernel


def kernel(feat_table, W1, b1, nodes, neigh_index):
    wt = W1.T.astype(jnp.float32)
    wx = jnp.concatenate([wt[:D, :], wt[D:, :] * (1.0 / DEG)], axis=1)
    b1row = b1.astype(jnp.float32).reshape(1, D)

    p_self, p_neigh = _tc_project(feat_table.astype(jnp.float32), wx, b1row)

    nodes_i = nodes.astype(jnp.int32)
    neigh_i = neigh_index.astype(jnp.int32).reshape(-1)
    return _make_sc_gather_sum()(p_self, p_neigh, nodes_i, neigh_i)


# speedup vs baseline: 6.1634x; 1.0210x over previous
import functools

import jax
import jax.numpy as jnp
from jax import lax
from jax.experimental import pallas as pl
from jax.experimental.pallas import tpu as pltpu
from jax.experimental.pallas import tpu_sc as plsc

N_NODES = 10000
DEG = 32
D = 128
B = 10000

NC = 2
NS = 16
NW = NC * NS
LANES = 16

NPAD = 10112
SROWS = NPAD // NS
RPW = 320
RCHUNK = 4
NCHUNKS = RPW // RCHUNK
NPAIRS = NCHUNKS // 2


def _tc_project(feat_table, wx, b1row):
    def body(f_ref, w_ref, b_ref, ps_ref, pn_ref):
        f = f_ref[...]
        w = w_ref[...]
        dn = (((1,), (1,)), ((), ()))
        ps_ref[...] = (
            lax.dot_general(f, w[:, :D], dn, preferred_element_type=jnp.float32)
            + b_ref[...]
        )
        pn_ref[...] = lax.dot_general(
            f, w[:, D:], dn, preferred_element_type=jnp.float32) * (1.0 / DEG)

    blk = 1264
    return pl.pallas_call(
        body,
        grid=(NPAD // blk,),
        in_specs=[
            pl.BlockSpec((blk, D), lambda i: (i, 0)),
            pl.BlockSpec((D, 2 * D), lambda i: (0, 0)),
            pl.BlockSpec((1, D), lambda i: (0, 0)),
        ],
        out_specs=[
            pl.BlockSpec((blk, D), lambda i: (i, 0)),
            pl.BlockSpec((blk, D), lambda i: (i, 0)),
        ],
        out_shape=[
            jax.ShapeDtypeStruct((NPAD, D), jnp.float32),
            jax.ShapeDtypeStruct((NPAD, D), jnp.float32),
        ],
    )(feat_table, wx, b1row)


def _make_sc_gather_sum():
    mesh = plsc.VectorSubcoreMesh(core_axis_name="c", subcore_axis_name="s")

    @functools.partial(
        pl.kernel,
        mesh=mesh,
        out_type=jax.ShapeDtypeStruct((B, D), jnp.float32),
        scratch_types=[
            pltpu.VMEM_SHARED((NPAD, D), jnp.float32),
            pltpu.VMEM((RPW,), jnp.int32),
            pltpu.VMEM((RPW * DEG,), jnp.int32),
            pltpu.VMEM((2 * RCHUNK, D), jnp.float32),
            pltpu.VMEM((2 * RCHUNK, D), jnp.float32),
            pltpu.VMEM((RCHUNK * DEG, D), jnp.float32),
            pltpu.VMEM((RCHUNK * DEG, D), jnp.float32),
            pltpu.VMEM((RCHUNK, D), jnp.float32),
            pltpu.VMEM((RCHUNK, D), jnp.float32),
            pltpu.SemaphoreType.DMA,
            pltpu.SemaphoreType.DMA,
            pltpu.SemaphoreType.DMA,
            pltpu.SemaphoreType.DMA,
            pltpu.SemaphoreType.DMA,
        ],
    )
    def sc_kernel(ps_hbm, pn_hbm, nodes_hbm, neigh_hbm, out_hbm,
                  shared_tbl, idxs_all, idxn_all, rs0, rs1, rn0, rn1, ov0, ov1,
                  ssem, nsem0, nsem1, osem0, osem1):
        cid = lax.axis_index("c")
        sid = lax.axis_index("s")
        wid = sid * NC + cid
        base = lax.min(wid * RPW, B - RPW)

        rows_s = (rs0, rs1)
        rows_n = (rn0, rn1)
        out_v = (ov0, ov1)
        nsem = (nsem0, nsem1)
        osem = (osem0, osem1)

        st0 = pltpu.async_copy(pn_hbm.at[pl.ds(sid * SROWS, SROWS)],
                               shared_tbl.at[pl.ds(sid * SROWS, SROWS)], ssem)
        st1 = pltpu.async_copy(nodes_hbm.at[pl.ds(base, RPW)], idxs_all, ssem)
        st2 = pltpu.async_copy(neigh_hbm.at[pl.ds(base * DEG, RPW * DEG)],
                               idxn_all, ssem)
        st0.wait()
        st1.wait()
        st2.wait()
        plsc.subcore_barrier()

        def issue_self(p, pslot):
            pltpu.async_copy(
                ps_hbm.at[idxs_all.at[pl.ds(p * 2 * RCHUNK, 2 * RCHUNK)]],
                rows_s[pslot], ssem)

        def wait_self(pslot):
            pltpu.make_async_copy(
                ps_hbm.at[pl.ds(0, 2 * RCHUNK)], rows_s[pslot], ssem).wait()

        def issue_neigh(c, slot):
            pltpu.async_copy(
                shared_tbl.at[idxn_all.at[pl.ds(c * (RCHUNK * DEG), RCHUNK * DEG)]],
                rows_n[slot], nsem[slot])

        def wait_neigh(slot):
            pltpu.make_async_copy(
                pn_hbm.at[pl.ds(0, RCHUNK * DEG)], rows_n[slot],
                nsem[slot]).wait()

        def compute_chunk(nslot, pslot, srow0):
            rn = rows_n[nslot]
            rs = rows_s[pslot]
            ov = out_v[nslot]

            def row(r, carry2):
                for c in range(D // LANES):
                    sl = pl.ds(c * LANES, LANES)
                    a0 = rs[srow0 + r, sl] + rn[r * DEG + 0, sl]
                    a1 = rn[r * DEG + 1, sl]
                    a2 = rn[r * DEG + 2, sl]
                    a3 = rn[r * DEG + 3, sl]
                    for j in range(4, DEG, 4):
                        a0 = a0 + rn[r * DEG + j, sl]
                        a1 = a1 + rn[r * DEG + j + 1, sl]
                        a2 = a2 + rn[r * DEG + j + 2, sl]
                        a3 = a3 + rn[r * DEG + j + 3, sl]
                    acc = (a0 + a1) + (a2 + a3)
                    ov[r, sl] = jnp.maximum(acc, 0.0)
                return carry2

            lax.fori_loop(0, RCHUNK, row, 0)

        def step(c, i, nslot, pslot, srow0, issue_self_next):
            @pl.when(c + 1 < NCHUNKS)
            def _():
                issue_neigh(c + 1, 1 - nslot)

            wait_neigh(nslot)

            if issue_self_next:
                @pl.when(i + 1 < NPAIRS)
                def _():
                    issue_self(i + 1, 1 - pslot)

            @pl.when(i > 0)
            def _():
                pltpu.make_async_copy(
                    out_v[nslot], out_hbm.at[pl.ds(0, RCHUNK)],
                    osem[nslot]).wait()

            compute_chunk(nslot, pslot, srow0)
            pltpu.async_copy(
                out_v[nslot], out_hbm.at[pl.ds(base + c * RCHUNK, RCHUNK)],
                osem[nslot])

        issue_self(0, 0)
        issue_neigh(0, 0)

        def pair2(i2, carry):
            p0 = 2 * i2
            p1 = 2 * i2 + 1
            wait_self(0)
            step(2 * p0, p0, 0, 0, 0, issue_self_next=False)
            step(2 * p0 + 1, p0, 1, 0, RCHUNK, issue_self_next=True)
            wait_self(1)
            step(2 * p1, p1, 0, 1, 0, issue_self_next=False)
            step(2 * p1 + 1, p1, 1, 1, RCHUNK, issue_self_next=True)
            return carry

        lax.fori_loop(0, NPAIRS // 2, pair2, 0)

        pltpu.make_async_copy(
            out_v[0], out_hbm.at[pl.ds(0, RCHUNK)], osem[0]).wait()
        pltpu.make_async_copy(
            out_v[1], out_hbm.at[pl.ds(0, RCHUNK)], osem[1]).wait()

    return sc_kernel


def kernel(feat_table, W1, b1, nodes, neigh_index):
    wx = W1.astype(jnp.float32)
    b1row = b1.astype(jnp.float32).reshape(1, D)

    p_self, p_neigh = _tc_project(feat_table.astype(jnp.float32), wx, b1row)

    nodes_i = nodes.astype(jnp.int32)
    neigh_i = neigh_index.astype(jnp.int32).reshape(-1)
    return _make_sc_gather_sum()(p_self, p_neigh, nodes_i, neigh_i)
